# baseline (speedup 1.0000x reference)
import functools

import jax
import jax.numpy as jnp
from jax import lax
from jax.experimental import pallas as pl
from jax.experimental.pallas import tpu as pltpu
from jax.experimental.pallas import tpu_sc as plsc

_OUT = 7
_G = 14
_C = 256
_RTOT = 400
_NW = 32
_NITER = 13
_ROW = _C * _OUT * _OUT


def _f32(x):
    return x.astype(jnp.float32)


def _ifloor(x):
    return (x - 0.5).astype(jnp.int32)


def _roi_meta(r, meta_v):
    mv = meta_v[r, :]
    lev = jnp.clip(mv[4].astype(jnp.int32), 0, 3)
    b = jnp.where(r < 200, 0, 1)
    W = 128 >> lev
    scale = jnp.where(lev == 0, 0.25,
            jnp.where(lev == 1, 0.125,
            jnp.where(lev == 2, 0.0625, jnp.float32(0.03125))))
    off = jnp.where(lev == 0, 0,
          jnp.where(lev == 1, 32768,
          jnp.where(lev == 2, 40960, 43008)))
    pixbase = off + b * (W * W)
    y1 = mv[0] * scale
    x1 = mv[1] * scale
    y2 = mv[2] * scale
    x2 = mv[3] * scale
    bin_h = jnp.maximum(y2 - y1, 1.0) * jnp.float32(1.0 / 7.0)
    bin_w = jnp.maximum(x2 - x1, 1.0) * jnp.float32(1.0 / 7.0)

    def _first(c1, binv):
        s0 = jnp.clip(c1 + binv * 0.25, 0.0, _f32(W - 1))
        return jnp.minimum(_ifloor(s0), W - 2)

    ystart = _first(y1, bin_h)
    xstart = _first(x1, bin_w)
    return (y1, x1, bin_h, bin_w, ystart, xstart, W, pixbase)


def _write_idx(m, lane, idx_ref):
    _, _, _, _, ystart, xstart, W, pixbase = m
    Wm1 = W - 1
    for v in range(4):
        q = lane + v * 16
        row = jnp.minimum(ystart + (q >> 3), Wm1)
        col = jnp.minimum(xstart + (q & 7), Wm1)
        idx_ref[pl.ds(v * 16, 16)] = pixbase + row * W + col


def _dot8(w, v):
    t = [w[i] * v[i] for i in range(8)]
    return ((t[0] + t[1]) + (t[2] + t[3])) + ((t[4] + t[5]) + (t[6] + t[7]))


def _compute(r, m, patch_v, xp_v, ob_v, out, sem):
    y1, x1, bin_h, bin_w, ystart, xstart, W, _ = m
    Wf = _f32(W - 1)
    xs = [jnp.clip(x1 + bin_w * ((s + 0.5) * 0.5), 0.0, Wf) for s in range(_G)]
    ys = [jnp.clip(y1 + bin_h * ((s + 0.5) * 0.5), 0.0, Wf) for s in range(_G)]
    nqy = jnp.minimum(_ifloor(ys[_G - 1]), W - 2) - ystart + 2

    def _hat(v, X):
        return jnp.maximum(1.0 - jnp.abs(v - X), 0.0)

    wx = [[None] * 8 for _ in range(_OUT)]
    wy = [[None] * 8 for _ in range(_OUT)]
    for q in range(8):
        X = _f32(xstart + q)
        Y = _f32(ystart + q)
        for p in range(_OUT):
            wx[p][q] = _hat(xs[2 * p], X) + _hat(xs[2 * p + 1], X)
            wy[p][q] = (_hat(ys[2 * p], Y) + _hat(ys[2 * p + 1], Y)) * 0.25

    def _chunk(ch, carry):
        c16 = ch * 16

        def _p1(qy, c1):
            row = [patch_v[qy * 8 + qx, pl.ds(c16, 16)] for qx in range(8)]
            for px in range(_OUT):
                xp_v[qy * 7 + px, :] = _dot8(wx[px], row)
            return c1

        lax.fori_loop(0, nqy, _p1, 0)

        for px in range(_OUT):
            col = [xp_v[qy * 7 + px, :] for qy in range(8)]
            for py in range(_OUT):
                ob_v[pl.ds((py * 7 + px) * _C + c16, 16)] = _dot8(wy[py], col)
        return carry

    lax.fori_loop(0, 16, _chunk, 0, unroll=2)
    pltpu.sync_copy(ob_v, out.at[pl.ds(r * _ROW, _ROW)])


_mesh = plsc.VectorSubcoreMesh(core_axis_name="c", subcore_axis_name="s")


@functools.partial(
    pl.kernel,
    out_type=jax.ShapeDtypeStruct((_RTOT * _ROW,), jnp.float32),
    mesh=_mesh,
    scratch_types=[
        pltpu.VMEM((_RTOT, 16), jnp.float32),
        pltpu.VMEM((2, 64), jnp.int32),
        pltpu.VMEM((2, 64, _C), jnp.float32),
        pltpu.VMEM((56, 16), jnp.float32),
        pltpu.VMEM((_ROW,), jnp.float32),
        pltpu.SemaphoreType.DMA,
    ],
)
def _pooler_sc(table, meta, out, meta_v, idx_v, patch_v, xp_v, ob_v, sem):
    wid = lax.axis_index("s") * 2 + lax.axis_index("c")
    pltpu.sync_copy(meta, meta_v)
    lane = lax.broadcasted_iota(jnp.int32, (16,), 0)

    zv = jnp.zeros((16,), jnp.float32)
    for i in range(56):
        xp_v[i, :] = zv

    m0 = _roi_meta(wid, meta_v)
    _write_idx(m0, lane, idx_v.at[0])
    pltpu.async_copy(table.at[idx_v.at[0]], patch_v.at[0], sem).wait()

    def _loop(t, m):
        r = wid + t * _NW
        rn = jnp.minimum(r + _NW, _RTOT - 1)
        cur = t & 1
        nxt = (t + 1) & 1
        mn = _roi_meta(rn, meta_v)
        _write_idx(mn, lane, idx_v.at[nxt])
        pltpu.async_copy(table.at[idx_v.at[nxt]], patch_v.at[nxt], sem)

        @pl.when(r < _RTOT)
        def _():
            _compute(r, m, patch_v.at[cur], xp_v, ob_v, out, sem)

        pltpu.make_async_copy(table.at[idx_v.at[nxt]], patch_v.at[nxt],
                              sem).wait()
        return mn

    lax.fori_loop(0, _NITER, _loop, m0)


def kernel(feat0, feat1, feat2, feat3, boxes):
    feats = (feat0, feat1, feat2, feat3)
    table = jnp.concatenate(
        [jnp.transpose(f, (0, 2, 3, 1)).reshape(-1, _C) for f in feats], axis=0)
    flat = boxes.reshape(-1, 4)
    areas_n = ((flat[:, 2] - flat[:, 0]) * (flat[:, 3] - flat[:, 1])) / (512.0 * 512.0)
    s = jnp.sqrt(areas_n)
    lev = jnp.round(4.0 + jnp.log2(s + 1e-6)).astype(jnp.int32) + 2
    lev = jnp.clip(lev, 0, 6)
    lev = jnp.where(s > 0.65, 5, lev)
    meta = jnp.zeros((_RTOT, 16), jnp.float32)
    meta = meta.at[:, 0:4].set(flat)
    meta = meta.at[:, 4].set(lev.astype(jnp.float32))
    out = _pooler_sc(table, meta)
    return jnp.transpose(out.reshape(_RTOT, _OUT * _OUT, _C), (0, 2, 1)) \
              .reshape(_RTOT, _C, _OUT, _OUT)


# speedup vs baseline: 1.0267x; 1.0267x over previous
import functools

import jax
import jax.numpy as jnp
from jax import lax
from jax.experimental import pallas as pl
from jax.experimental.pallas import tpu as pltpu
from jax.experimental.pallas import tpu_sc as plsc

_OUT = 7
_G = 14
_C = 256
_RTOT = 400
_NW = 32
_NITER = 13
_ROW = _C * _OUT * _OUT


def _f32(x):
    return x.astype(jnp.float32)


def _ifloor(x):
    return (x - 0.5).astype(jnp.int32)


def _roi_meta(r, meta_v):
    mv = meta_v[r, :]
    lev = jnp.clip(mv[4].astype(jnp.int32), 0, 3)
    b = jnp.where(r < 200, 0, 1)
    W = 128 >> lev
    scale = jnp.where(lev == 0, 0.25,
            jnp.where(lev == 1, 0.125,
            jnp.where(lev == 2, 0.0625, jnp.float32(0.03125))))
    off = jnp.where(lev == 0, 0,
          jnp.where(lev == 1, 32768,
          jnp.where(lev == 2, 40960, 43008)))
    pixbase = off + b * (W * W)
    y1 = mv[0] * scale
    x1 = mv[1] * scale
    y2 = mv[2] * scale
    x2 = mv[3] * scale
    bin_h = jnp.maximum(y2 - y1, 1.0) * jnp.float32(1.0 / 7.0)
    bin_w = jnp.maximum(x2 - x1, 1.0) * jnp.float32(1.0 / 7.0)

    def _first(c1, binv):
        s0 = jnp.clip(c1 + binv * 0.25, 0.0, _f32(W - 1))
        return jnp.minimum(_ifloor(s0), W - 2)

    ystart = _first(y1, bin_h)
    xstart = _first(x1, bin_w)
    return (y1, x1, bin_h, bin_w, ystart, xstart, W, pixbase)


def _write_idx(m, lane, idx_ref):
    _, _, _, _, ystart, xstart, W, pixbase = m
    Wm1 = W - 1
    for v in range(4):
        q = lane + v * 16
        row = jnp.minimum(ystart + (q >> 3), Wm1)
        col = jnp.minimum(xstart + (q & 7), Wm1)
        idx_ref[pl.ds(v * 16, 16)] = pixbase + row * W + col


def _dot8(w, v):
    t = [w[i] * v[i] for i in range(8)]
    return ((t[0] + t[1]) + (t[2] + t[3])) + ((t[4] + t[5]) + (t[6] + t[7]))


def _compute(r, m, patch_v, xp_v, ob_v, out, sem):
    y1, x1, bin_h, bin_w, ystart, xstart, W, _ = m
    Wf = _f32(W - 1)
    xs = [jnp.clip(x1 + bin_w * ((s + 0.5) * 0.5), 0.0, Wf) for s in range(_G)]
    ys = [jnp.clip(y1 + bin_h * ((s + 0.5) * 0.5), 0.0, Wf) for s in range(_G)]
    nqy = jnp.minimum(_ifloor(ys[_G - 1]), W - 2) - ystart + 2

    def _hat(v, X):
        return jnp.maximum(1.0 - jnp.abs(v - X), 0.0)

    wx = [[None] * 8 for _ in range(_OUT)]
    wy = [[None] * 8 for _ in range(_OUT)]
    for q in range(8):
        X = _f32(xstart + q)
        Y = _f32(ystart + q)
        for p in range(_OUT):
            wx[p][q] = _hat(xs[2 * p], X) + _hat(xs[2 * p + 1], X)
            wy[p][q] = (_hat(ys[2 * p], Y) + _hat(ys[2 * p + 1], Y)) * 0.25

    def _chunk(ch, carry):
        c16 = ch * 16

        def _p1(qy, c1):
            row = [patch_v[qy * 8 + qx, pl.ds(c16, 16)] for qx in range(8)]
            for px in range(_OUT):
                xp_v[qy * 7 + px, :] = _dot8(wx[px], row)
            return c1

        lax.fori_loop(0, nqy, _p1, 0)

        for px in range(_OUT):
            col = [xp_v[qy * 7 + px, :] for qy in range(8)]
            for py in range(_OUT):
                ob_v[pl.ds((py * 7 + px) * _C + c16, 16)] = _dot8(wy[py], col)
        return carry

    lax.fori_loop(0, 16, _chunk, 0)
    pltpu.sync_copy(ob_v, out.at[pl.ds(r * _ROW, _ROW)])


_mesh = plsc.VectorSubcoreMesh(core_axis_name="c", subcore_axis_name="s")


@functools.partial(
    pl.kernel,
    out_type=jax.ShapeDtypeStruct((_RTOT * _ROW,), jnp.float32),
    mesh=_mesh,
    scratch_types=[
        pltpu.VMEM((_RTOT, 16), jnp.float32),
        pltpu.VMEM((2, 64), jnp.int32),
        pltpu.VMEM((2, 64, _C), jnp.float32),
        pltpu.VMEM((56, 16), jnp.float32),
        pltpu.VMEM((_ROW,), jnp.float32),
        pltpu.SemaphoreType.DMA,
    ],
)
def _pooler_sc(table, meta, out, meta_v, idx_v, patch_v, xp_v, ob_v, sem):
    wid = lax.axis_index("s") * 2 + lax.axis_index("c")
    pltpu.sync_copy(meta, meta_v)
    lane = lax.broadcasted_iota(jnp.int32, (16,), 0)

    zv = jnp.zeros((16,), jnp.float32)
    for i in range(56):
        xp_v[i, :] = zv

    m0 = _roi_meta(wid, meta_v)
    _write_idx(m0, lane, idx_v.at[0])
    pltpu.async_copy(table.at[idx_v.at[0]], patch_v.at[0], sem).wait()

    def _loop(t, m):
        r = wid + t * _NW
        rn = jnp.minimum(r + _NW, _RTOT - 1)
        cur = t & 1
        nxt = (t + 1) & 1
        mn = _roi_meta(rn, meta_v)
        _write_idx(mn, lane, idx_v.at[nxt])
        pltpu.async_copy(table.at[idx_v.at[nxt]], patch_v.at[nxt], sem)

        @pl.when(r < _RTOT)
        def _():
            _compute(r, m, patch_v.at[cur], xp_v, ob_v, out, sem)

        pltpu.make_async_copy(table.at[idx_v.at[nxt]], patch_v.at[nxt],
                              sem).wait()
        return mn

    lax.fori_loop(0, _NITER, _loop, m0)


def kernel(feat0, feat1, feat2, feat3, boxes):
    feats = (feat0, feat1, feat2, feat3)
    table = jnp.concatenate(
        [jnp.transpose(f, (0, 2, 3, 1)).reshape(-1, _C) for f in feats], axis=0)
    flat = boxes.reshape(-1, 4)
    areas_n = ((flat[:, 2] - flat[:, 0]) * (flat[:, 3] - flat[:, 1])) / (512.0 * 512.0)
    s = jnp.sqrt(areas_n)
    lev = jnp.round(4.0 + jnp.log2(s + 1e-6)).astype(jnp.int32) + 2
    lev = jnp.clip(lev, 0, 6)
    lev = jnp.where(s > 0.65, 5, lev)
    meta = jnp.zeros((_RTOT, 16), jnp.float32)
    meta = meta.at[:, 0:4].set(flat)
    meta = meta.at[:, 4].set(lev.astype(jnp.float32))
    out = _pooler_sc(table, meta)
    return jnp.transpose(out.reshape(_RTOT, _OUT * _OUT, _C), (0, 2, 1)) \
              .reshape(_RTOT, _C, _OUT, _OUT)


# speedup vs baseline: 1.0680x; 1.0401x over previous
import functools

import jax
import jax.numpy as jnp
from jax import lax
from jax.experimental import pallas as pl
from jax.experimental.pallas import tpu as pltpu
from jax.experimental.pallas import tpu_sc as plsc

_OUT = 7
_G = 14
_C = 256
_RTOT = 400
_NW = 32
_NITER = 13
_ROW = _C * _OUT * _OUT


def _f32(x):
    return x.astype(jnp.float32)


def _ifloor(x):
    return (x - 0.5).astype(jnp.int32)


def _roi_meta(r, meta_v):
    mv = meta_v[r, :]
    lev = jnp.clip(mv[4].astype(jnp.int32), 0, 3)
    b = jnp.where(r < 200, 0, 1)
    W = 128 >> lev
    scale = jnp.where(lev == 0, 0.25,
            jnp.where(lev == 1, 0.125,
            jnp.where(lev == 2, 0.0625, jnp.float32(0.03125))))
    off = jnp.where(lev == 0, 0,
          jnp.where(lev == 1, 32768,
          jnp.where(lev == 2, 40960, 43008)))
    pixbase = off + b * (W * W)
    y1 = mv[0] * scale
    x1 = mv[1] * scale
    y2 = mv[2] * scale
    x2 = mv[3] * scale
    bin_h = jnp.maximum(y2 - y1, 1.0) * jnp.float32(1.0 / 7.0)
    bin_w = jnp.maximum(x2 - x1, 1.0) * jnp.float32(1.0 / 7.0)

    def _first(c1, binv):
        s0 = jnp.clip(c1 + binv * 0.25, 0.0, _f32(W - 1))
        return jnp.minimum(_ifloor(s0), W - 2)

    ystart = _first(y1, bin_h)
    xstart = _first(x1, bin_w)
    return (y1, x1, bin_h, bin_w, ystart, xstart, W, pixbase)


def _write_idx(m, lane, idx_ref):
    _, _, _, _, ystart, xstart, W, pixbase = m
    Wm1 = W - 1
    for v in range(4):
        q = lane + v * 16
        row = jnp.minimum(ystart + (q >> 3), Wm1)
        col = jnp.minimum(xstart + (q & 7), Wm1)
        idx_ref[pl.ds(v * 16, 16)] = pixbase + row * W + col


def _dot8(w, v):
    acc = w[0] * v[0]
    for i in range(1, 8):
        acc = acc + w[i] * v[i]
    return acc


def _compute(r, m, patch_v, xp_v, ob_v, out, sem):
    y1, x1, bin_h, bin_w, ystart, xstart, W, _ = m
    Wf = _f32(W - 1)
    xs = [jnp.clip(x1 + bin_w * ((s + 0.5) * 0.5), 0.0, Wf) for s in range(_G)]
    ys = [jnp.clip(y1 + bin_h * ((s + 0.5) * 0.5), 0.0, Wf) for s in range(_G)]
    nqy = jnp.minimum(_ifloor(ys[_G - 1]), W - 2) - ystart + 2

    def _hat(v, X):
        return jnp.maximum(1.0 - jnp.abs(v - X), 0.0)

    wx = [[None] * 8 for _ in range(_OUT)]
    wy = [[None] * 8 for _ in range(_OUT)]
    for q in range(8):
        X = _f32(xstart + q)
        Y = _f32(ystart + q)
        for p in range(_OUT):
            wx[p][q] = _hat(xs[2 * p], X) + _hat(xs[2 * p + 1], X)
            wy[p][q] = (_hat(ys[2 * p], Y) + _hat(ys[2 * p + 1], Y)) * 0.25

    def _chunk(ch, carry):
        c16 = ch * 16

        def _p1(qy, c1):
            row = [patch_v[qy * 8 + qx, pl.ds(c16, 16)] for qx in range(8)]
            for px in range(_OUT):
                xp_v[qy * 7 + px, :] = _dot8(wx[px], row)
            return c1

        lax.fori_loop(0, nqy, _p1, 0)

        for px in range(_OUT):
            col = [xp_v[qy * 7 + px, :] for qy in range(8)]
            for py in range(_OUT):
                ob_v[pl.ds((py * 7 + px) * _C + c16, 16)] = _dot8(wy[py], col)
        return carry

    lax.fori_loop(0, 16, _chunk, 0)
    pltpu.sync_copy(ob_v, out.at[pl.ds(r * _ROW, _ROW)])


_mesh = plsc.VectorSubcoreMesh(core_axis_name="c", subcore_axis_name="s")


@functools.partial(
    pl.kernel,
    out_type=jax.ShapeDtypeStruct((_RTOT * _ROW,), jnp.float32),
    mesh=_mesh,
    scratch_types=[
        pltpu.VMEM((_RTOT, 16), jnp.float32),
        pltpu.VMEM((2, 64), jnp.int32),
        pltpu.VMEM((2, 64, _C), jnp.float32),
        pltpu.VMEM((56, 16), jnp.float32),
        pltpu.VMEM((_ROW,), jnp.float32),
        pltpu.SemaphoreType.DMA,
    ],
)
def _pooler_sc(table, meta, out, meta_v, idx_v, patch_v, xp_v, ob_v, sem):
    wid = lax.axis_index("s") * 2 + lax.axis_index("c")
    pltpu.sync_copy(meta, meta_v)
    lane = lax.broadcasted_iota(jnp.int32, (16,), 0)

    zv = jnp.zeros((16,), jnp.float32)
    for i in range(56):
        xp_v[i, :] = zv

    m0 = _roi_meta(wid, meta_v)
    _write_idx(m0, lane, idx_v.at[0])
    pltpu.async_copy(table.at[idx_v.at[0]], patch_v.at[0], sem).wait()

    def _loop(t, m):
        r = wid + t * _NW
        rn = jnp.minimum(r + _NW, _RTOT - 1)
        cur = t & 1
        nxt = (t + 1) & 1
        mn = _roi_meta(rn, meta_v)
        _write_idx(mn, lane, idx_v.at[nxt])
        pltpu.async_copy(table.at[idx_v.at[nxt]], patch_v.at[nxt], sem)

        @pl.when(r < _RTOT)
        def _():
            _compute(r, m, patch_v.at[cur], xp_v, ob_v, out, sem)

        pltpu.make_async_copy(table.at[idx_v.at[nxt]], patch_v.at[nxt],
                              sem).wait()
        return mn

    lax.fori_loop(0, _NITER, _loop, m0)


def kernel(feat0, feat1, feat2, feat3, boxes):
    feats = (feat0, feat1, feat2, feat3)
    table = jnp.concatenate(
        [jnp.transpose(f, (0, 2, 3, 1)).reshape(-1, _C) for f in feats], axis=0)
    flat = boxes.reshape(-1, 4)
    areas_n = ((flat[:, 2] - flat[:, 0]) * (flat[:, 3] - flat[:, 1])) / (512.0 * 512.0)
    s = jnp.sqrt(areas_n)
    lev = jnp.round(4.0 + jnp.log2(s + 1e-6)).astype(jnp.int32) + 2
    lev = jnp.clip(lev, 0, 6)
    lev = jnp.where(s > 0.65, 5, lev)
    meta = jnp.zeros((_RTOT, 16), jnp.float32)
    meta = meta.at[:, 0:4].set(flat)
    meta = meta.at[:, 4].set(lev.astype(jnp.float32))
    out = _pooler_sc(table, meta)
    return jnp.transpose(out.reshape(_RTOT, _OUT * _OUT, _C), (0, 2, 1)) \
              .reshape(_RTOT, _C, _OUT, _OUT)
